# baseline (speedup 1.0000x reference)
import functools

import jax
import jax.numpy as jnp
from jax import lax
from jax.experimental import pallas as pl
from jax.experimental.pallas import tpu as pltpu
from jax.experimental.pallas import tpu_sc as plsc

B = 64
D = 8732
C = 21
CP = 24
O = 8
L = 16
DP = 8736
NCH = DP // L
NW = 32
BPW = B // NW
SLAB = 672
CPS = SLAB // L
NSLAB_FULL = 12
TAIL_ROWS = D - NSLAB_FULL * SLAB
LN2 = 0.6931471805599453
BS_ITERS = 26


def _logf(x):
    b = lax.bitcast_convert_type(x, jnp.int32)
    e = jnp.bitwise_and(lax.shift_right_logical(b, 23), 0xFF) - 127
    mb = jnp.bitwise_or(jnp.bitwise_and(b, 0x7FFFFF), 0x3F800000)
    m = lax.bitcast_convert_type(mb, jnp.float32)
    r = (m - 1.0) / (m + 1.0)
    r2 = r * r
    p = 2.0 * r * (1.0 + r2 * (1.0 / 3.0 + r2 * (1.0 / 5.0 + r2 * (1.0 / 7.0 + r2 * (1.0 / 9.0)))))
    return e.astype(jnp.float32) * LN2 + p


def _smooth_l1(x):
    ax = jnp.abs(x)
    return jnp.where(ax < 1.0, 0.5 * x * x, ax - 0.5)


def _body(loc_hbm, conf_hbm, dbox_hbm, tgt_hbm, out_hbm,
          dbox_v, conf_v, locb_v, tgt_v, btv_v, bti_v, ce_v, out_v):
    wid = lax.axis_index("s") * 2 + lax.axis_index("c")
    iota_i = lax.iota(jnp.int32, L)
    iota_f = iota_i.astype(jnp.float32)
    z16 = jnp.zeros((L,), jnp.float32)

    pltpu.sync_copy(dbox_hbm, dbox_v)

    w_np = z16
    w_l1 = z16
    w_cp = z16
    w_num = jnp.float32(0.0)
    w_k = jnp.float32(0.0)

    for bi in range(BPW):
        b = wid * BPW + bi
        pltpu.sync_copy(tgt_hbm.at[b], tgt_v.at[pl.ds(0, O * 5)])
        tvecs = [tgt_v[pl.ds(16 * i, 16)] for i in range(3)]

        def _tg(idx):
            return tvecs[idx // 16][idx % 16]

        tx1 = [_tg(j * 5 + 0) for j in range(O)]
        ty1 = [_tg(j * 5 + 1) for j in range(O)]
        tx2 = [_tg(j * 5 + 2) for j in range(O)]
        ty2 = [_tg(j * 5 + 3) for j in range(O)]
        tlab = [_tg(j * 5 + 4) for j in range(O)]
        area_a = [(tx2[j] - tx1[j]) * (ty2[j] - ty1[j]) for j in range(O)]

        def _iou_chunk(base):
            cx = dbox_v[0, pl.ds(base, L)]
            cy = dbox_v[1, pl.ds(base, L)]
            w = dbox_v[2, pl.ds(base, L)]
            h = dbox_v[3, pl.ds(base, L)]
            xmn = cx - w / 2.0
            ymn = cy - h / 2.0
            xmx = cx + w / 2.0
            ymx = cy + h / 2.0
            area_b = (xmx - xmn) * (ymx - ymn)
            ious = []
            for j in range(O):
                iw = jnp.maximum(jnp.minimum(xmx, tx2[j]) - jnp.maximum(xmn, tx1[j]), 0.0)
                ih = jnp.maximum(jnp.minimum(ymx, ty2[j]) - jnp.maximum(ymn, ty1[j]), 0.0)
                inter = iw * ih
                ious.append(inter / (area_a[j] + area_b - inter))
            return ious

        def pass_a(ch, carry):
            rmax = list(carry[0:O])
            ridx = list(carry[O:2 * O])
            base = ch * L
            gidx_f = (ch * L + iota_i).astype(jnp.float32)
            ious = _iou_chunk(base)
            bv = jnp.full((L,), -1.0, jnp.float32)
            bj = z16
            for j in range(O):
                u = ious[j] > rmax[j]
                rmax[j] = jnp.where(u, ious[j], rmax[j])
                ridx[j] = jnp.where(u, gidx_f, ridx[j])
                u2 = ious[j] > bv
                bv = jnp.where(u2, ious[j], bv)
                bj = jnp.where(u2, jnp.float32(j), bj)
            btv_v[pl.ds(base, L)] = bv
            bti_v[pl.ds(base, L)] = bj
            return tuple(rmax) + tuple(ridx)

        init = tuple(jnp.full((L,), -1.0, jnp.float32) for _ in range(O)) + tuple(z16 for _ in range(O))
        fin = lax.fori_loop(0, NCH, pass_a, init)
        pbest = []
        for j in range(O):
            m = jnp.max(fin[j])
            pbest.append(jnp.min(jnp.where(fin[j] == m, fin[O + j], 1e9)))

        def chunk_bc(s, c, carry):
            acc_np, acc_l1, acc_cp, acc_mx = carry
            gch = s * CPS + c
            base_g = gch * L
            base_l = c * L
            gidx_f = (gch * L + iota_i).astype(jnp.float32)
            valid = gidx_f < jnp.float32(D)
            bv = btv_v[pl.ds(base_g, L)]
            bj = bti_v[pl.ds(base_g, L)]
            for j in range(O):
                f = gidx_f == pbest[j]
                bv = jnp.where(f, 2.0, bv)
                bj = jnp.where(f, jnp.float32(j), bj)
            lab = z16
            mx1 = z16
            my1 = z16
            mx2 = z16
            my2 = z16
            for j in range(O):
                sel = bj == jnp.float32(j)
                lab = jnp.where(sel, tlab[j], lab)
                mx1 = jnp.where(sel, tx1[j], mx1)
                my1 = jnp.where(sel, ty1[j], my1)
                mx2 = jnp.where(sel, tx2[j], mx2)
                my2 = jnp.where(sel, ty2[j], my2)
            conf_lab = jnp.where(bv < 0.5, 0.0, lab + 1.0)
            conf_lab = jnp.where(valid, conf_lab, 0.0)
            pos = conf_lab > 0.5
            pcx = dbox_v[0, pl.ds(base_g, L)]
            pcy = dbox_v[1, pl.ds(base_g, L)]
            pw = dbox_v[2, pl.ds(base_g, L)]
            ph = dbox_v[3, pl.ds(base_g, L)]
            gcx = ((mx1 + mx2) / 2.0 - pcx) / (0.1 * pw)
            gcy = ((my1 + my2) / 2.0 - pcy) / (0.1 * ph)
            gw = _logf((mx2 - mx1) / pw) / 0.2
            gh = _logf((my2 - my1) / ph) / 0.2
            lrow = base_l + iota_i
            zi = iota_i * 0
            l0 = plsc.load_gather(locb_v, [lrow, zi])
            l1 = plsc.load_gather(locb_v, [lrow, zi + 1])
            l2 = plsc.load_gather(locb_v, [lrow, zi + 2])
            l3 = plsc.load_gather(locb_v, [lrow, zi + 3])
            sl1 = (_smooth_l1(l0 - gcx) + _smooth_l1(l1 - gcy)
                   + _smooth_l1(l2 - gw) + _smooth_l1(l3 - gh))
            acc_np = acc_np + jnp.where(pos, 1.0, 0.0)
            acc_l1 = acc_l1 + jnp.where(pos, sl1, 0.0)
            lrow = base_l + iota_i
            vs = [conf_v[k, pl.ds(base_l, L)] for k in range(C)]
            m = vs[0]
            for k in range(1, C):
                m = jnp.maximum(m, vs[k])
            sexp = jnp.exp(vs[0] - m)
            for k in range(1, C):
                sexp = sexp + jnp.exp(vs[k] - m)
            lse = m + _logf(sexp)
            labi = conf_lab.astype(jnp.int32)
            picked = plsc.load_gather(conf_v, [labi, lrow])
            ce = lse - picked
            ce = jnp.where(valid, ce, 0.0)
            acc_cp = acc_cp + jnp.where(pos, ce, 0.0)
            cem = jnp.where(pos, 0.0, ce)
            ce_v[pl.ds(base_g, L)] = cem
            acc_mx = jnp.maximum(acc_mx, cem)
            return acc_np, acc_l1, acc_cp, acc_mx

        def full_slab(s, carry):
            pltpu.sync_copy(conf_hbm.at[b, :, pl.ds(s * SLAB, SLAB)], conf_v)
            pltpu.sync_copy(loc_hbm.at[b, pl.ds(s * SLAB, SLAB), :], locb_v)
            return lax.fori_loop(0, CPS, lambda c, cc: chunk_bc(s, c, cc), carry)

        carry = (z16, z16, z16, z16)
        carry = lax.fori_loop(0, NSLAB_FULL, full_slab, carry)
        pltpu.sync_copy(conf_hbm.at[b, :, pl.ds(NSLAB_FULL * SLAB, SLAB)], conf_v)
        pltpu.sync_copy(loc_hbm.at[b, pl.ds(NSLAB_FULL * SLAB, SLAB), :], locb_v)
        carry = lax.fori_loop(0, CPS,
                              lambda c, cc: chunk_bc(jnp.int32(NSLAB_FULL), c, cc), carry)
        acc_np, acc_l1, acc_cp, acc_mx = carry

        n_pos = jnp.sum(acc_np)
        kk = jnp.clip(3.0 * n_pos, 20.0, jnp.float32(D))
        maxce = jnp.max(acc_mx)

        UNR = 8

        def bs_step(_, lohi):
            lo, hi = lohi
            mid = 0.5 * (lo + hi)

            def cnt_blk(blk, a):
                base = blk * (UNR * L)
                for u in range(UNR):
                    v = ce_v[pl.ds(base + u * L, L)]
                    a = a + jnp.where(v > mid, 1.0, 0.0)
                return a

            a = lax.fori_loop(0, NCH // UNR, cnt_blk, z16)
            for ch in range((NCH // UNR) * UNR, NCH):
                v = ce_v[pl.ds(ch * L, L)]
                a = a + jnp.where(v > mid, 1.0, 0.0)
            cnt = jnp.sum(a)
            gt = cnt > kk
            return jnp.where(gt, mid, lo), jnp.where(gt, hi, mid)

        lo, hi = lax.fori_loop(0, BS_ITERS, bs_step, (jnp.float32(0.0), maxce))

        def fin_blk(blk, a):
            sv, cv = a
            base = blk * (UNR * L)
            for u in range(UNR):
                v = ce_v[pl.ds(base + u * L, L)]
                g = v > hi
                sv = sv + jnp.where(g, v, 0.0)
                cv = cv + jnp.where(g, 1.0, 0.0)
            return sv, cv

        sv, cv = lax.fori_loop(0, NCH // UNR, fin_blk, (z16, z16))
        for ch in range((NCH // UNR) * UNR, NCH):
            v = ce_v[pl.ds(ch * L, L)]
            g = v > hi
            sv = sv + jnp.where(g, v, 0.0)
            cv = cv + jnp.where(g, 1.0, 0.0)
        num = jnp.sum(sv) + (kk - jnp.sum(cv)) * hi

        w_np = w_np + acc_np
        w_l1 = w_l1 + acc_l1
        w_cp = w_cp + acc_cp
        w_num = w_num + num
        w_k = w_k + kk

    out_v[0, :] = w_np
    out_v[1, :] = w_l1
    out_v[2, :] = w_cp
    out_v[3, :] = jnp.where(iota_i == 0, w_num, 0.0)
    out_v[4, :] = jnp.where(iota_i == 0, w_k, 0.0)
    pltpu.sync_copy(out_v, out_hbm.at[wid])


_mbl = functools.partial(
    pl.kernel,
    mesh=plsc.VectorSubcoreMesh(core_axis_name="c", subcore_axis_name="s"),
    out_type=jax.ShapeDtypeStruct((NW, 5, L), jnp.float32),
    compiler_params=pltpu.CompilerParams(use_tc_tiling_on_sc=False,
                                         needs_layout_passes=False),
    scratch_types=[
        pltpu.VMEM((4, DP), jnp.float32),
        pltpu.VMEM((CP, SLAB), jnp.float32),
        pltpu.VMEM((SLAB, 8), jnp.float32),
        pltpu.VMEM((48,), jnp.float32),
        pltpu.VMEM((DP,), jnp.float32),
        pltpu.VMEM((DP,), jnp.float32),
        pltpu.VMEM((DP,), jnp.float32),
        pltpu.VMEM((5, L), jnp.float32),
    ],
)(_body)


def kernel(loc_data, conf_data, dbox_list, targets):
    dbox_t = jnp.pad(dbox_list, ((0, DP - D), (0, 0))).T
    conf_t = jnp.pad(jnp.transpose(conf_data, (0, 2, 1)),
                     ((0, 0), (0, CP - C), (0, DP - D)))
    loc_p = jnp.pad(loc_data, ((0, 0), (0, DP - D), (0, 4)))
    tgt_flat = targets.reshape(B, O * 5)
    out = _mbl(loc_p, conf_t, dbox_t, tgt_flat)
    s = jnp.sum(out, axis=(0, 2))
    n_pos = s[0]
    loss_l = jnp.nan_to_num(s[1] / (4.0 * n_pos))
    loss_c_pos = jnp.nan_to_num(s[2] / n_pos)
    loss_c_neg = jnp.nan_to_num(s[3] / s[4])
    return loss_l, loss_c_pos + loss_c_neg, loss_c_pos, loss_c_neg


# speedup vs baseline: 2.8489x; 2.8489x over previous
import functools

import jax
import jax.numpy as jnp
from jax import lax
from jax.experimental import pallas as pl
from jax.experimental.pallas import tpu as pltpu
from jax.experimental.pallas import tpu_sc as plsc

B = 64
D = 8732
C = 21
CP = 24
O = 8
L = 16
DP = 8736
NCH = DP // L
NW = 32
BPW = B // NW
SLAB = 672
CPS = SLAB // L
NSLAB_FULL = 12
TAIL_ROWS = D - NSLAB_FULL * SLAB
LN2 = 0.6931471805599453
BS_ITERS = 26


def _logf(x):
    b = lax.bitcast_convert_type(x, jnp.int32)
    e = jnp.bitwise_and(lax.shift_right_logical(b, 23), 0xFF) - 127
    mb = jnp.bitwise_or(jnp.bitwise_and(b, 0x7FFFFF), 0x3F800000)
    m = lax.bitcast_convert_type(mb, jnp.float32)
    r = (m - 1.0) / (m + 1.0)
    r2 = r * r
    p = 2.0 * r * (1.0 + r2 * (1.0 / 3.0 + r2 * (1.0 / 5.0 + r2 * (1.0 / 7.0 + r2 * (1.0 / 9.0)))))
    return e.astype(jnp.float32) * LN2 + p


def _smooth_l1(x):
    ax = jnp.abs(x)
    return jnp.where(ax < 1.0, 0.5 * x * x, ax - 0.5)


def _body(loc_hbm, conf_hbm, dbox_hbm, tgt_hbm, out_hbm,
          dbox_v, conf_v, locb_v, tgt_v, btv_v, bti_v, ce_v, out_v):
    wid = lax.axis_index("s") * 2 + lax.axis_index("c")
    iota_i = lax.iota(jnp.int32, L)
    iota_f = iota_i.astype(jnp.float32)
    z16 = jnp.zeros((L,), jnp.float32)

    pltpu.sync_copy(dbox_hbm, dbox_v)

    w_np = z16
    w_l1 = z16
    w_cp = z16
    w_num = jnp.float32(0.0)
    w_k = jnp.float32(0.0)

    for bi in range(BPW):
        b = wid * BPW + bi
        pltpu.sync_copy(tgt_hbm.at[b], tgt_v.at[pl.ds(0, O * 5)])
        tvecs = [tgt_v[pl.ds(16 * i, 16)] for i in range(3)]

        def _tg(idx):
            return tvecs[idx // 16][idx % 16]

        tx1 = [_tg(j * 5 + 0) for j in range(O)]
        ty1 = [_tg(j * 5 + 1) for j in range(O)]
        tx2 = [_tg(j * 5 + 2) for j in range(O)]
        ty2 = [_tg(j * 5 + 3) for j in range(O)]
        tlab = [_tg(j * 5 + 4) for j in range(O)]
        area_a = [(tx2[j] - tx1[j]) * (ty2[j] - ty1[j]) for j in range(O)]

        def _iou_chunk(base):
            cx = dbox_v[0, pl.ds(base, L)]
            cy = dbox_v[1, pl.ds(base, L)]
            w = dbox_v[2, pl.ds(base, L)]
            h = dbox_v[3, pl.ds(base, L)]
            xmn = cx - w / 2.0
            ymn = cy - h / 2.0
            xmx = cx + w / 2.0
            ymx = cy + h / 2.0
            area_b = (xmx - xmn) * (ymx - ymn)
            ious = []
            for j in range(O):
                iw = jnp.maximum(jnp.minimum(xmx, tx2[j]) - jnp.maximum(xmn, tx1[j]), 0.0)
                ih = jnp.maximum(jnp.minimum(ymx, ty2[j]) - jnp.maximum(ymn, ty1[j]), 0.0)
                inter = iw * ih
                ious.append(inter / (area_a[j] + area_b - inter))
            return ious

        def pass_a(ch, carry):
            rmax = list(carry[0:O])
            ridx = list(carry[O:2 * O])
            base = ch * L
            gidx_f = (ch * L + iota_i).astype(jnp.float32)
            ious = _iou_chunk(base)
            bv = jnp.full((L,), -1.0, jnp.float32)
            bj = z16
            for j in range(O):
                u = ious[j] > rmax[j]
                rmax[j] = jnp.where(u, ious[j], rmax[j])
                ridx[j] = jnp.where(u, gidx_f, ridx[j])
                u2 = ious[j] > bv
                bv = jnp.where(u2, ious[j], bv)
                bj = jnp.where(u2, jnp.float32(j), bj)
            btv_v[pl.ds(base, L)] = bv
            bti_v[pl.ds(base, L)] = bj
            return tuple(rmax) + tuple(ridx)

        init = tuple(jnp.full((L,), -1.0, jnp.float32) for _ in range(O)) + tuple(z16 for _ in range(O))
        fin = lax.fori_loop(0, NCH, pass_a, init)
        pbest = []
        for j in range(O):
            m = jnp.max(fin[j])
            pbest.append(jnp.min(jnp.where(fin[j] == m, fin[O + j], 1e9)))

        def chunk_bc(s, c, carry):
            acc_np, acc_l1, acc_cp, acc_mx = carry
            gch = s * CPS + c
            base_g = gch * L
            base_l = c * L
            gidx_f = (gch * L + iota_i).astype(jnp.float32)
            valid = gidx_f < jnp.float32(D)
            bv = btv_v[pl.ds(base_g, L)]
            bj = bti_v[pl.ds(base_g, L)]
            for j in range(O):
                f = gidx_f == pbest[j]
                bv = jnp.where(f, 2.0, bv)
                bj = jnp.where(f, jnp.float32(j), bj)
            lab = z16
            mx1 = z16
            my1 = z16
            mx2 = z16
            my2 = z16
            for j in range(O):
                sel = bj == jnp.float32(j)
                lab = jnp.where(sel, tlab[j], lab)
                mx1 = jnp.where(sel, tx1[j], mx1)
                my1 = jnp.where(sel, ty1[j], my1)
                mx2 = jnp.where(sel, tx2[j], mx2)
                my2 = jnp.where(sel, ty2[j], my2)
            conf_lab = jnp.where(bv < 0.5, 0.0, lab + 1.0)
            conf_lab = jnp.where(valid, conf_lab, 0.0)
            pos = conf_lab > 0.5
            pcx = dbox_v[0, pl.ds(base_g, L)]
            pcy = dbox_v[1, pl.ds(base_g, L)]
            pw = dbox_v[2, pl.ds(base_g, L)]
            ph = dbox_v[3, pl.ds(base_g, L)]
            gcx = ((mx1 + mx2) / 2.0 - pcx) / (0.1 * pw)
            gcy = ((my1 + my2) / 2.0 - pcy) / (0.1 * ph)
            gw = _logf((mx2 - mx1) / pw) / 0.2
            gh = _logf((my2 - my1) / ph) / 0.2
            lrow4 = (base_l + iota_i) * 4
            l0 = plsc.load_gather(locb_v, [lrow4])
            l1 = plsc.load_gather(locb_v, [lrow4 + 1])
            l2 = plsc.load_gather(locb_v, [lrow4 + 2])
            l3 = plsc.load_gather(locb_v, [lrow4 + 3])
            sl1 = (_smooth_l1(l0 - gcx) + _smooth_l1(l1 - gcy)
                   + _smooth_l1(l2 - gw) + _smooth_l1(l3 - gh))
            acc_np = acc_np + jnp.where(pos, 1.0, 0.0)
            acc_l1 = acc_l1 + jnp.where(pos, sl1, 0.0)
            lrow = base_l + iota_i
            vs = [conf_v[k, pl.ds(base_l, L)] for k in range(C)]
            m = vs[0]
            for k in range(1, C):
                m = jnp.maximum(m, vs[k])
            sexp = jnp.exp(vs[0] - m)
            for k in range(1, C):
                sexp = sexp + jnp.exp(vs[k] - m)
            lse = m + _logf(sexp)
            labi = conf_lab.astype(jnp.int32)
            picked = plsc.load_gather(conf_v, [labi, lrow])
            ce = lse - picked
            ce = jnp.where(valid, ce, 0.0)
            acc_cp = acc_cp + jnp.where(pos, ce, 0.0)
            cem = jnp.where(pos, 0.0, ce)
            ce_v[pl.ds(base_g, L)] = cem
            acc_mx = jnp.maximum(acc_mx, cem)
            return acc_np, acc_l1, acc_cp, acc_mx

        def full_slab(s, carry):
            pltpu.sync_copy(conf_hbm.at[b, :, pl.ds(s * SLAB, SLAB)], conf_v)
            pltpu.sync_copy(loc_hbm.at[b, pl.ds(s * (SLAB * 4), SLAB * 4)], locb_v)
            return lax.fori_loop(0, CPS, lambda c, cc: chunk_bc(s, c, cc), carry)

        carry = (z16, z16, z16, z16)
        carry = lax.fori_loop(0, NSLAB_FULL, full_slab, carry)
        pltpu.sync_copy(conf_hbm.at[b, :, pl.ds(NSLAB_FULL * SLAB, SLAB)], conf_v)
        pltpu.sync_copy(loc_hbm.at[b, pl.ds(NSLAB_FULL * SLAB * 4, TAIL_ROWS * 4)],
                        locb_v.at[pl.ds(0, TAIL_ROWS * 4)])
        carry = lax.fori_loop(0, CPS,
                              lambda c, cc: chunk_bc(jnp.int32(NSLAB_FULL), c, cc), carry)
        acc_np, acc_l1, acc_cp, acc_mx = carry

        n_pos = jnp.sum(acc_np)
        kk = jnp.clip(3.0 * n_pos, 20.0, jnp.float32(D))
        maxce = jnp.max(acc_mx)

        UNR = 8

        def bs_step(_, lohi):
            lo, hi = lohi
            mid = 0.5 * (lo + hi)

            def cnt_blk(blk, a):
                base = blk * (UNR * L)
                for u in range(UNR):
                    v = ce_v[pl.ds(base + u * L, L)]
                    a = a + jnp.where(v > mid, 1.0, 0.0)
                return a

            a = lax.fori_loop(0, NCH // UNR, cnt_blk, z16)
            for ch in range((NCH // UNR) * UNR, NCH):
                v = ce_v[pl.ds(ch * L, L)]
                a = a + jnp.where(v > mid, 1.0, 0.0)
            cnt = jnp.sum(a)
            gt = cnt > kk
            return jnp.where(gt, mid, lo), jnp.where(gt, hi, mid)

        lo, hi = lax.fori_loop(0, BS_ITERS, bs_step, (jnp.float32(0.0), maxce))

        def fin_blk(blk, a):
            sv, cv = a
            base = blk * (UNR * L)
            for u in range(UNR):
                v = ce_v[pl.ds(base + u * L, L)]
                g = v > hi
                sv = sv + jnp.where(g, v, 0.0)
                cv = cv + jnp.where(g, 1.0, 0.0)
            return sv, cv

        sv, cv = lax.fori_loop(0, NCH // UNR, fin_blk, (z16, z16))
        for ch in range((NCH // UNR) * UNR, NCH):
            v = ce_v[pl.ds(ch * L, L)]
            g = v > hi
            sv = sv + jnp.where(g, v, 0.0)
            cv = cv + jnp.where(g, 1.0, 0.0)
        num = jnp.sum(sv) + (kk - jnp.sum(cv)) * hi

        w_np = w_np + acc_np
        w_l1 = w_l1 + acc_l1
        w_cp = w_cp + acc_cp
        w_num = w_num + num
        w_k = w_k + kk

    out_v[0, :] = w_np
    out_v[1, :] = w_l1
    out_v[2, :] = w_cp
    out_v[3, :] = jnp.where(iota_i == 0, w_num, 0.0)
    out_v[4, :] = jnp.where(iota_i == 0, w_k, 0.0)
    pltpu.sync_copy(out_v, out_hbm.at[wid])


_mbl = functools.partial(
    pl.kernel,
    mesh=plsc.VectorSubcoreMesh(core_axis_name="c", subcore_axis_name="s"),
    out_type=jax.ShapeDtypeStruct((NW, 5, L), jnp.float32),
    compiler_params=pltpu.CompilerParams(use_tc_tiling_on_sc=False,
                                         needs_layout_passes=False),
    scratch_types=[
        pltpu.VMEM((4, DP), jnp.float32),
        pltpu.VMEM((CP, SLAB), jnp.float32),
        pltpu.VMEM((SLAB * 4,), jnp.float32),
        pltpu.VMEM((48,), jnp.float32),
        pltpu.VMEM((DP,), jnp.float32),
        pltpu.VMEM((DP,), jnp.float32),
        pltpu.VMEM((DP,), jnp.float32),
        pltpu.VMEM((5, L), jnp.float32),
    ],
)(_body)


def kernel(loc_data, conf_data, dbox_list, targets):
    dbox_t = jnp.pad(dbox_list, ((0, DP - D), (0, 0))).T
    conf_t = jnp.pad(jnp.transpose(conf_data, (0, 2, 1)),
                     ((0, 0), (0, CP - C), (0, DP - D)))
    loc_flat = loc_data.reshape(B, D * 4)
    tgt_flat = targets.reshape(B, O * 5)
    out = _mbl(loc_flat, conf_t, dbox_t, tgt_flat)
    s = jnp.sum(out, axis=(0, 2))
    n_pos = s[0]
    loss_l = jnp.nan_to_num(s[1] / (4.0 * n_pos))
    loss_c_pos = jnp.nan_to_num(s[2] / n_pos)
    loss_c_neg = jnp.nan_to_num(s[3] / s[4])
    return loss_l, loss_c_pos + loss_c_neg, loss_c_pos, loss_c_neg
